# baseline (speedup 1.0000x reference)
import functools

import jax
import jax.numpy as jnp
from jax import lax
from jax.experimental import pallas as pl
from jax.experimental.pallas import tpu as pltpu
from jax.experimental.pallas import tpu_sc as plsc

NUM_TABLES = 26
DIM = 32
VOCAB = 100000
BATCH = 16384
NBUF = 3


VOCAB_P = 100096
NGRP = DIM // 8
NTILE = VOCAB_P // 128
CBLK = 34


def _detile_kernel(x_ref, o_ref):
    for g in range(NGRP):
        o_ref[:, 8 * g:8 * (g + 1)] = (
            x_ref[g].transpose(0, 2, 1).reshape(CBLK * 128, 8))


_detile = pl.pallas_call(
    _detile_kernel,
    grid=(NTILE // CBLK,),
    in_specs=[pl.BlockSpec((NGRP, CBLK, 8, 128), lambda j: (0, j, 0, 0))],
    out_specs=pl.BlockSpec((CBLK * 128, DIM), lambda j: (j, 0)),
    out_shape=jax.ShapeDtypeStruct((VOCAB_P, DIM), jnp.float32),
)


def _to_rowmajor(w):
    wp = jnp.pad(w.T, ((0, 0), (0, VOCAB_P - VOCAB)))
    w4 = wp.reshape(NGRP, 8, NTILE, 128).transpose(0, 2, 1, 3)
    return _detile(w4)


def _build():
    info = plsc.get_sparse_core_info()
    nc, ns = info.num_cores, info.num_subcores
    nw = nc * ns
    bpw = BATCH // nw
    mesh = plsc.VectorSubcoreMesh(core_axis_name="c", subcore_axis_name="s")

    @functools.partial(
        pl.kernel,
        mesh=mesh,
        out_type=jax.ShapeDtypeStruct((BATCH, NUM_TABLES * DIM), jnp.float32),
        scratch_types=(
            [pltpu.VMEM((NUM_TABLES, bpw), jnp.int32)]
            + [pltpu.VMEM((bpw, DIM), jnp.float32) for _ in range(NBUF)]
            + [pltpu.SemaphoreType.DMA for _ in range(2 * NBUF + 1)]
        ),
        compiler_params=pltpu.CompilerParams(use_tc_tiling_on_sc=False),
    )
    def k(*refs):
        idx_refs = refs[:NUM_TABLES]
        tab_refs = refs[NUM_TABLES:2 * NUM_TABLES]
        out = refs[2 * NUM_TABLES]
        rest = refs[2 * NUM_TABLES + 1:]
        idx_all = rest[0]
        bufs = rest[1:1 + NBUF]
        gsems = rest[1 + NBUF:1 + 2 * NBUF]
        wsems = rest[1 + 2 * NBUF:1 + 3 * NBUF]
        isem = rest[1 + 3 * NBUF]

        wid = lax.axis_index("s") * nc + lax.axis_index("c")
        base = wid * bpw

        idescs = [
            pltpu.async_copy(
                idx_refs[t].at[pl.ds(base, bpw)], idx_all.at[t], isem)
            for t in range(NUM_TABLES)
        ]
        for d in idescs:
            d.wait()

        def gather(t, s):
            return pltpu.async_copy(
                tab_refs[t].at[idx_all.at[t]], bufs[s], gsems[s])

        def write(t, s):
            return pltpu.async_copy(
                bufs[s],
                out.at[pl.ds(base, bpw), pl.ds(t * DIM, DIM)],
                wsems[s])

        gd = [None] * NBUF
        wd = [None] * NBUF
        for t in range(min(NBUF, NUM_TABLES)):
            gd[t % NBUF] = gather(t, t % NBUF)
        for t in range(NUM_TABLES):
            s = t % NBUF
            gd[s].wait()
            wd[s] = write(t, s)
            nt = t + NBUF
            if nt < NUM_TABLES:
                wd[s].wait()
                wd[s] = None
                gd[s] = gather(nt, s)
        for s in range(NBUF):
            if wd[s] is not None:
                wd[s].wait()

    return k


_gather_concat = _build()


def kernel(f0, f1, f2, f3, f4, f5, f6, f7, f8, f9, f10, f11, f12, f13, f14,
           f15, f16, f17, f18, f19, f20, f21, f22, f23, f24, f25,
           W_f0, W_f1, W_f2, W_f3, W_f4, W_f5, W_f6, W_f7, W_f8, W_f9, W_f10,
           W_f11, W_f12, W_f13, W_f14, W_f15, W_f16, W_f17, W_f18, W_f19,
           W_f20, W_f21, W_f22, W_f23, W_f24, W_f25):
    idx = [f0, f1, f2, f3, f4, f5, f6, f7, f8, f9, f10, f11, f12, f13, f14,
           f15, f16, f17, f18, f19, f20, f21, f22, f23, f24, f25]
    tabs = [W_f0, W_f1, W_f2, W_f3, W_f4, W_f5, W_f6, W_f7, W_f8, W_f9,
            W_f10, W_f11, W_f12, W_f13, W_f14, W_f15, W_f16, W_f17, W_f18,
            W_f19, W_f20, W_f21, W_f22, W_f23, W_f24, W_f25]
    tabs_rm = [_to_rowmajor(w) for w in tabs]
    return _gather_concat(*idx, *tabs_rm)


# speedup vs baseline: 3.1698x; 3.1698x over previous
import functools

import jax
import jax.numpy as jnp
from jax import lax
from jax.experimental import pallas as pl
from jax.experimental.pallas import tpu as pltpu
from jax.experimental.pallas import tpu_sc as plsc

NUM_TABLES = 26
DIM = 32
VOCAB = 100000
BATCH = 16384
NBUF = 3


VOCAB_P = 100096
NGRP = DIM // 8
NTILE = VOCAB_P // 128
CBLK = 34


def _detile_kernel(x_ref, o_ref):
    for g in range(NGRP):
        o_ref[:, 8 * g:8 * (g + 1)] = (
            x_ref[g].transpose(0, 2, 1).reshape(CBLK * 128, 8))


_detile = pl.pallas_call(
    _detile_kernel,
    grid=(NTILE // CBLK,),
    in_specs=[pl.BlockSpec((NGRP, CBLK, 8, 128), lambda j: (0, j, 0, 0))],
    out_specs=pl.BlockSpec((CBLK * 128, DIM), lambda j: (j, 0)),
    out_shape=jax.ShapeDtypeStruct((VOCAB_P, DIM), jnp.float32),
)


def _to_rowmajor(w):
    wp = jnp.pad(w.T, ((0, 0), (0, VOCAB_P - VOCAB)))
    w4 = wp.reshape(NGRP, 8, NTILE, 128).transpose(0, 2, 1, 3)
    return _detile(w4)


def _build():
    info = plsc.get_sparse_core_info()
    nc, ns = info.num_cores, info.num_subcores
    nw = nc * ns
    bpw = BATCH // nw
    mesh = plsc.VectorSubcoreMesh(core_axis_name="c", subcore_axis_name="s")

    @functools.partial(
        pl.kernel,
        mesh=mesh,
        out_type=jax.ShapeDtypeStruct((BATCH, NUM_TABLES * DIM), jnp.float32),
        scratch_types=(
            [pltpu.VMEM((NUM_TABLES, bpw), jnp.int32)]
            + [pltpu.VMEM((bpw, DIM), jnp.float32) for _ in range(NBUF)]
            + [pltpu.SemaphoreType.DMA for _ in range(2 * NBUF + 1)]
        ),
        compiler_params=pltpu.CompilerParams(use_tc_tiling_on_sc=False),
    )
    def k(*refs):
        idx_refs = refs[:NUM_TABLES]
        tab_refs = refs[NUM_TABLES:2 * NUM_TABLES]
        out = refs[2 * NUM_TABLES]
        rest = refs[2 * NUM_TABLES + 1:]
        idx_all = rest[0]
        bufs = rest[1:1 + NBUF]
        gsems = rest[1 + NBUF:1 + 2 * NBUF]
        wsems = rest[1 + 2 * NBUF:1 + 3 * NBUF]
        isem = rest[1 + 3 * NBUF]

        wid = lax.axis_index("s") * nc + lax.axis_index("c")
        base = wid * bpw

        idescs = [
            pltpu.async_copy(
                idx_refs[t].at[pl.ds(base, bpw)], idx_all.at[t], isem)
            for t in range(NUM_TABLES)
        ]
        for d in idescs:
            d.wait()

        def gather(t, s):
            return pltpu.async_copy(
                tab_refs[t].at[idx_all.at[t]], bufs[s], gsems[s])

        def write(t, s):
            return pltpu.async_copy(
                bufs[s],
                out.at[pl.ds(base, bpw), pl.ds(t * DIM, DIM)],
                wsems[s])

        gd = [None] * NBUF
        wd = [None] * NBUF
        for t in range(min(NBUF, NUM_TABLES)):
            gd[t % NBUF] = gather(t, t % NBUF)
        for t in range(NUM_TABLES):
            s = t % NBUF
            gd[s].wait()
            wd[s] = write(t, s)
            nt = t + NBUF
            if nt < NUM_TABLES:
                wd[s].wait()
                wd[s] = None
                gd[s] = gather(nt, s)
        for s in range(NBUF):
            if wd[s] is not None:
                wd[s].wait()

    return k


_gather_concat = _build()


def kernel(f0, f1, f2, f3, f4, f5, f6, f7, f8, f9, f10, f11, f12, f13, f14,
           f15, f16, f17, f18, f19, f20, f21, f22, f23, f24, f25,
           W_f0, W_f1, W_f2, W_f3, W_f4, W_f5, W_f6, W_f7, W_f8, W_f9, W_f10,
           W_f11, W_f12, W_f13, W_f14, W_f15, W_f16, W_f17, W_f18, W_f19,
           W_f20, W_f21, W_f22, W_f23, W_f24, W_f25):
    idx = [f0, f1, f2, f3, f4, f5, f6, f7, f8, f9, f10, f11, f12, f13, f14,
           f15, f16, f17, f18, f19, f20, f21, f22, f23, f24, f25]
    tabs = [W_f0, W_f1, W_f2, W_f3, W_f4, W_f5, W_f6, W_f7, W_f8, W_f9,
            W_f10, W_f11, W_f12, W_f13, W_f14, W_f15, W_f16, W_f17, W_f18,
            W_f19, W_f20, W_f21, W_f22, W_f23, W_f24, W_f25]
    return _gather_concat(*idx, *tabs)
